# baseline (speedup 1.0000x reference)
import functools
import math

import jax
import jax.numpy as jnp
from jax import lax
from jax.experimental import pallas as pl
from jax.experimental.pallas import tpu as pltpu
from jax.experimental.pallas import tpu_sc as plsc

_PI = math.pi
_A = math.sqrt(4.0 * _PI)
_C1 = _A * math.sqrt(3.0 / (4.0 * _PI))
_C2XY = _A * 0.5 * math.sqrt(15.0 / _PI)
_C2Z2 = _A * 0.25 * math.sqrt(5.0 / _PI)
_C2XX = _A * 0.25 * math.sqrt(15.0 / _PI)
_C30 = _A * 0.25 * math.sqrt(35.0 / (2.0 * _PI))
_C31 = _A * 0.5 * math.sqrt(105.0 / _PI)
_C32 = _A * 0.25 * math.sqrt(21.0 / (2.0 * _PI))
_C33 = _A * 0.25 * math.sqrt(7.0 / _PI)
_C34 = _A * 0.25 * math.sqrt(105.0 / _PI)

_NW = 32
_LANES = 16


def _rsqrt_newton(r2):
    bits = lax.bitcast_convert_type(r2, jnp.int32)
    seed = jnp.int32(0x5F3759DF) - lax.shift_right_logical(bits, 1)
    y = lax.bitcast_convert_type(seed, jnp.float32)
    h = jnp.float32(0.5) * r2
    for _ in range(4):
        y = y * (jnp.float32(1.5) - h * y * y)
    return y


def _sh_components(x, y, z):
    xx = x * x
    yy = y * y
    zz = z * z
    xy = x * y
    yz = y * z
    xz = x * z
    t8 = xx - yy
    t5z = jnp.float32(5.0) * zz - jnp.float32(1.0)
    sh = [None] * 16
    sh[0] = jnp.full((_LANES,), 1.0, jnp.float32)
    sh[1] = jnp.float32(_C1) * y
    sh[2] = jnp.float32(_C1) * z
    sh[3] = jnp.float32(_C1) * x
    sh[4] = jnp.float32(_C2XY) * xy
    sh[5] = jnp.float32(_C2XY) * yz
    sh[6] = jnp.float32(_C2Z2) * (jnp.float32(3.0) * zz - jnp.float32(1.0))
    sh[7] = jnp.float32(_C2XY) * xz
    sh[8] = jnp.float32(_C2XX) * t8
    sh[9] = jnp.float32(_C30) * y * (jnp.float32(3.0) * xx - yy)
    sh[10] = jnp.float32(_C31) * xy * z
    sh[11] = jnp.float32(_C32) * y * t5z
    sh[12] = jnp.float32(_C33) * z * (t5z - jnp.float32(2.0))
    sh[13] = jnp.float32(_C32) * x * t5z
    sh[14] = jnp.float32(_C34) * z * t8
    sh[15] = jnp.float32(_C30) * x * (xx - jnp.float32(3.0) * yy)
    return sh


@functools.lru_cache(maxsize=None)
def _make_sc_fn(N, P):
    assert P % _NW == 0
    PW = P // _NW
    B = None
    for cand in (2000, 1000, 400, 80, 16):
        if PW % cand == 0:
            B = cand
            break
    assert B is not None, PW
    NBLK = PW // B
    NG = B // _LANES

    mesh = plsc.VectorSubcoreMesh(core_axis_name="c", subcore_axis_name="s")
    f32 = jnp.float32
    i32 = jnp.int32

    dbl = lambda *a: [pltpu.VMEM(*a), pltpu.VMEM(*a)]

    @functools.partial(
        pl.kernel,
        out_type=[
            jax.ShapeDtypeStruct((P,), f32),
            jax.ShapeDtypeStruct((P * 16,), f32),
        ],
        mesh=mesh,
        compiler_params=pltpu.CompilerParams(
            needs_layout_passes=False, use_tc_tiling_on_sc=False),
        scratch_types=[
            dbl((2 * B,), i32),
            dbl((3 * B,), i32),
            dbl((2 * B,), f32),
            dbl((2 * B,), f32),
            dbl((2 * B,), f32),
            dbl((B,), f32),
            dbl((B * 16,), f32),
            pltpu.VMEM((16,), f32),
            pltpu.VMEM((16,), i32),
            [pltpu.SemaphoreType.DMA, pltpu.SemaphoreType.DMA],
            [pltpu.SemaphoreType.DMA, pltpu.SemaphoreType.DMA],
            [pltpu.SemaphoreType.DMA, pltpu.SemaphoreType.DMA],
        ],
    )
    def sc_fn(px_hbm, py_hbm, pz_hbm, pv_hbm, sv_hbm, cell_hbm, off_hbm,
              r_hbm, sh_hbm, pv_v, sv_v, xg_v, yg_v, zg_v, r_v, sh_v,
              cell_v, off_v, semIn, semGat, semOut):
        wid = lax.axis_index("c") * 16 + lax.axis_index("s")
        base0 = wid * PW

        pltpu.sync_copy(cell_hbm, cell_v)
        pltpu.sync_copy(off_hbm, off_v)
        cv = cell_v[pl.ds(0, 16)]
        c00 = cv[0]
        c01 = cv[1]
        c02 = cv[2]
        c10 = cv[3]
        c11 = cv[4]
        c12 = cv[5]
        c20 = cv[6]
        c21 = cv[7]
        c22 = cv[8]
        off = off_v[pl.ds(0, 16)][0]

        iota = lax.broadcasted_iota(i32, (_LANES,), 0)
        cols = [jnp.full((_LANES,), m, i32) for m in range(16)]

        def issue_idx(t, p):
            base = base0 + t * B
            pltpu.async_copy(pv_hbm.at[pl.ds(base * 2, 2 * B)], pv_v[p], semIn[p])
            pltpu.async_copy(sv_hbm.at[pl.ds(base * 3, 3 * B)], sv_v[p], semIn[p])

        def drain_idx(p):
            pltpu.make_async_copy(pv_hbm.at[pl.ds(0, 2 * B)], pv_v[p], semIn[p]).wait()
            pltpu.make_async_copy(sv_hbm.at[pl.ds(0, 3 * B)], sv_v[p], semIn[p]).wait()

        def issue_gather(p):
            @pl.when(off != 0)
            def _():
                def add_off(g, c):
                    sl = pl.ds(g * _LANES, _LANES)
                    pv_v[p][sl] = pv_v[p][sl] + off
                    return c
                lax.fori_loop(0, 2 * NG, add_off, 0)
            pltpu.async_copy(px_hbm.at[pv_v[p]], xg_v[p], semGat[p])
            pltpu.async_copy(py_hbm.at[pv_v[p]], yg_v[p], semGat[p])
            pltpu.async_copy(pz_hbm.at[pv_v[p]], zg_v[p], semGat[p])

        def drain_gather(p):
            pltpu.make_async_copy(px_hbm.at[pv_v[p]], xg_v[p], semGat[p]).wait()
            pltpu.make_async_copy(py_hbm.at[pv_v[p]], yg_v[p], semGat[p]).wait()
            pltpu.make_async_copy(pz_hbm.at[pv_v[p]], zg_v[p], semGat[p]).wait()

        def issue_out(t, p):
            base = base0 + t * B
            pltpu.async_copy(r_v[p], r_hbm.at[pl.ds(base, B)], semOut[p])
            pltpu.async_copy(sh_v[p], sh_hbm.at[pl.ds(base * 16, B * 16)], semOut[p])

        def drain_out(p):
            pltpu.make_async_copy(r_v[p], r_hbm.at[pl.ds(0, B)], semOut[p]).wait()
            pltpu.make_async_copy(sh_v[p], sh_hbm.at[pl.ds(0, B * 16)], semOut[p]).wait()

        def compute(p):
            def grp(g, c):
                s = g * _LANES
                sl = pl.ds(s, _LANES)
                rows = s + iota
                rows16 = rows * 16
                l2 = rows * 2
                l3 = rows * 3
                xi = plsc.load_gather(xg_v[p], [l2])
                xj = plsc.load_gather(xg_v[p], [l2 + 1])
                yi = plsc.load_gather(yg_v[p], [l2])
                yj = plsc.load_gather(yg_v[p], [l2 + 1])
                zi = plsc.load_gather(zg_v[p], [l2])
                zj = plsc.load_gather(zg_v[p], [l2 + 1])
                svx = lax.convert_element_type(
                    plsc.load_gather(sv_v[p], [l3]), f32)
                svy = lax.convert_element_type(
                    plsc.load_gather(sv_v[p], [l3 + 1]), f32)
                svz = lax.convert_element_type(
                    plsc.load_gather(sv_v[p], [l3 + 2]), f32)
                dx = xj - xi + svx * c00 + svy * c10 + svz * c20
                dy = yj - yi + svx * c01 + svy * c11 + svz * c21
                dz = zj - zi + svx * c02 + svy * c12 + svz * c22
                r2 = dx * dx + dy * dy + dz * dz
                rinv = _rsqrt_newton(r2)
                r_v[p][sl] = r2 * rinv
                ux = dx * rinv
                uy = dy * rinv
                uz = dz * rinv
                sh = _sh_components(ux, uy, uz)
                for m in range(16):
                    plsc.store_scatter(sh_v[p], [rows16 + m], sh[m])
                return c
            lax.fori_loop(0, NG, grp, 0)

        issue_idx(0, 0)

        def step(t, p):
            q = 1 - p

            @pl.when(t < NBLK)
            def _():
                drain_idx(p)

                @pl.when(t + 1 < NBLK)
                def _():
                    issue_idx(t + 1, q)

                issue_gather(p)
                drain_gather(p)
                compute(p)
                issue_out(t, p)
                drain_out(p)

        def body2(o, carry):
            step(2 * o, 0)
            step(2 * o + 1, 1)
            return carry

        lax.fori_loop(0, (NBLK + 1) // 2, body2, 0)

    return sc_fn


def kernel(positions, cells, species, cell_shifts, centers, pairs,
           structure_centers, structure_pairs, structure_offsets):
    N = positions.shape[0]
    P = pairs.shape[0]
    px = positions[:, 0]
    py = positions[:, 1]
    pz = positions[:, 2]
    pairs_flat = pairs.reshape(-1)
    shifts_flat = cell_shifts.reshape(-1)
    cell16 = jnp.pad(cells[0].reshape(-1), (0, 7))
    off16 = jnp.pad(structure_offsets, (0, 16 - structure_offsets.shape[0]))
    r, sh_flat = _make_sc_fn(N, P)(px, py, pz, pairs_flat, shifts_flat,
                                   cell16, off16)
    return (r, sh_flat.reshape(P, 16))


# speedup vs baseline: 6.0515x; 6.0515x over previous
import functools
import math

import jax
import jax.numpy as jnp
from jax import lax
from jax.experimental import pallas as pl
from jax.experimental.pallas import tpu as pltpu
from jax.experimental.pallas import tpu_sc as plsc

_PI = math.pi
_A = math.sqrt(4.0 * _PI)
_C1 = _A * math.sqrt(3.0 / (4.0 * _PI))
_C2XY = _A * 0.5 * math.sqrt(15.0 / _PI)
_C2Z2 = _A * 0.25 * math.sqrt(5.0 / _PI)
_C2XX = _A * 0.25 * math.sqrt(15.0 / _PI)
_C30 = _A * 0.25 * math.sqrt(35.0 / (2.0 * _PI))
_C31 = _A * 0.5 * math.sqrt(105.0 / _PI)
_C32 = _A * 0.25 * math.sqrt(21.0 / (2.0 * _PI))
_C33 = _A * 0.25 * math.sqrt(7.0 / _PI)
_C34 = _A * 0.25 * math.sqrt(105.0 / _PI)

_NW = 32
_LANES = 16


def _rsqrt_newton(r2):
    bits = lax.bitcast_convert_type(r2, jnp.int32)
    seed = jnp.int32(0x5F3759DF) - lax.shift_right_logical(bits, 1)
    y = lax.bitcast_convert_type(seed, jnp.float32)
    h = jnp.float32(0.5) * r2
    for _ in range(4):
        y = y * (jnp.float32(1.5) - h * y * y)
    return y


def _sh_components(x, y, z):
    xx = x * x
    yy = y * y
    zz = z * z
    xy = x * y
    yz = y * z
    xz = x * z
    t8 = xx - yy
    t5z = jnp.float32(5.0) * zz - jnp.float32(1.0)
    sh = [None] * 16
    sh[0] = jnp.full((_LANES,), 1.0, jnp.float32)
    sh[1] = jnp.float32(_C1) * y
    sh[2] = jnp.float32(_C1) * z
    sh[3] = jnp.float32(_C1) * x
    sh[4] = jnp.float32(_C2XY) * xy
    sh[5] = jnp.float32(_C2XY) * yz
    sh[6] = jnp.float32(_C2Z2) * (jnp.float32(3.0) * zz - jnp.float32(1.0))
    sh[7] = jnp.float32(_C2XY) * xz
    sh[8] = jnp.float32(_C2XX) * t8
    sh[9] = jnp.float32(_C30) * y * (jnp.float32(3.0) * xx - yy)
    sh[10] = jnp.float32(_C31) * xy * z
    sh[11] = jnp.float32(_C32) * y * t5z
    sh[12] = jnp.float32(_C33) * z * (t5z - jnp.float32(2.0))
    sh[13] = jnp.float32(_C32) * x * t5z
    sh[14] = jnp.float32(_C34) * z * t8
    sh[15] = jnp.float32(_C30) * x * (xx - jnp.float32(3.0) * yy)
    return sh


@functools.lru_cache(maxsize=None)
def _make_sc_fn(N, P):
    assert P % _NW == 0
    PW = P // _NW
    B = None
    for cand in (2000, 1000, 400, 80, 16):
        if PW % cand == 0:
            B = cand
            break
    assert B is not None, PW
    NBLK = PW // B
    NG = B // _LANES

    mesh = plsc.VectorSubcoreMesh(core_axis_name="c", subcore_axis_name="s")
    f32 = jnp.float32
    i32 = jnp.int32

    dbl = lambda *a: [pltpu.VMEM(*a), pltpu.VMEM(*a)]

    @functools.partial(
        pl.kernel,
        out_type=[
            jax.ShapeDtypeStruct((P,), f32),
            jax.ShapeDtypeStruct((P * 16,), f32),
        ],
        mesh=mesh,
        compiler_params=pltpu.CompilerParams(
            needs_layout_passes=False, use_tc_tiling_on_sc=False),
        scratch_types=[
            dbl((B,), i32),
            dbl((B,), i32),
            dbl((B,), f32),
            dbl((B,), f32),
            dbl((B,), f32),
            dbl((B,), f32),
            dbl((B,), f32),
            dbl((B,), f32),
            dbl((B,), f32),
            dbl((B,), f32),
            dbl((B,), f32),
            dbl((B,), f32),
            dbl((B * 16,), f32),
            pltpu.VMEM((16,), f32),
            pltpu.VMEM((16,), i32),
            [pltpu.SemaphoreType.DMA, pltpu.SemaphoreType.DMA],
            [pltpu.SemaphoreType.DMA, pltpu.SemaphoreType.DMA],
            [pltpu.SemaphoreType.DMA, pltpu.SemaphoreType.DMA],
        ],
    )
    def sc_fn(px_hbm, py_hbm, pz_hbm, ii_hbm, jj_hbm, sx_hbm, sy_hbm,
              sz_hbm, cell_hbm, off_hbm, r_hbm, sh_hbm, ii_v, jj_v, sx_v,
              sy_v, sz_v, xi_v, yi_v, zi_v, xj_v, yj_v, zj_v, r_v, sh_v,
              cell_v, off_v, semIn, semGat, semOut):
        wid = lax.axis_index("c") * 16 + lax.axis_index("s")
        base0 = wid * PW

        pltpu.sync_copy(cell_hbm, cell_v)
        pltpu.sync_copy(off_hbm, off_v)
        cv = cell_v[pl.ds(0, 16)]
        c00 = cv[0]
        c01 = cv[1]
        c02 = cv[2]
        c10 = cv[3]
        c11 = cv[4]
        c12 = cv[5]
        c20 = cv[6]
        c21 = cv[7]
        c22 = cv[8]
        off = off_v[pl.ds(0, 16)][0]

        iota = lax.broadcasted_iota(i32, (_LANES,), 0)

        def issue_idx(t, p):
            base = base0 + t * B
            pltpu.async_copy(ii_hbm.at[pl.ds(base, B)], ii_v[p], semIn[p])
            pltpu.async_copy(jj_hbm.at[pl.ds(base, B)], jj_v[p], semIn[p])
            pltpu.async_copy(sx_hbm.at[pl.ds(base, B)], sx_v[p], semIn[p])
            pltpu.async_copy(sy_hbm.at[pl.ds(base, B)], sy_v[p], semIn[p])
            pltpu.async_copy(sz_hbm.at[pl.ds(base, B)], sz_v[p], semIn[p])

        def drain_idx(p):
            pltpu.make_async_copy(ii_hbm.at[pl.ds(0, B)], ii_v[p], semIn[p]).wait()
            pltpu.make_async_copy(jj_hbm.at[pl.ds(0, B)], jj_v[p], semIn[p]).wait()
            pltpu.make_async_copy(sx_hbm.at[pl.ds(0, B)], sx_v[p], semIn[p]).wait()
            pltpu.make_async_copy(sy_hbm.at[pl.ds(0, B)], sy_v[p], semIn[p]).wait()
            pltpu.make_async_copy(sz_hbm.at[pl.ds(0, B)], sz_v[p], semIn[p]).wait()

        def issue_gather(p):
            @pl.when(off != 0)
            def _():
                def add_off(g, c):
                    sl = pl.ds(g * _LANES, _LANES)
                    ii_v[p][sl] = ii_v[p][sl] + off
                    jj_v[p][sl] = jj_v[p][sl] + off
                    return c
                lax.fori_loop(0, NG, add_off, 0)
            pltpu.async_copy(px_hbm.at[ii_v[p]], xi_v[p], semGat[p])
            pltpu.async_copy(py_hbm.at[ii_v[p]], yi_v[p], semGat[p])
            pltpu.async_copy(pz_hbm.at[ii_v[p]], zi_v[p], semGat[p])
            pltpu.async_copy(px_hbm.at[jj_v[p]], xj_v[p], semGat[p])
            pltpu.async_copy(py_hbm.at[jj_v[p]], yj_v[p], semGat[p])
            pltpu.async_copy(pz_hbm.at[jj_v[p]], zj_v[p], semGat[p])

        def drain_gather(p):
            pltpu.make_async_copy(px_hbm.at[ii_v[p]], xi_v[p], semGat[p]).wait()
            pltpu.make_async_copy(py_hbm.at[ii_v[p]], yi_v[p], semGat[p]).wait()
            pltpu.make_async_copy(pz_hbm.at[ii_v[p]], zi_v[p], semGat[p]).wait()
            pltpu.make_async_copy(px_hbm.at[jj_v[p]], xj_v[p], semGat[p]).wait()
            pltpu.make_async_copy(py_hbm.at[jj_v[p]], yj_v[p], semGat[p]).wait()
            pltpu.make_async_copy(pz_hbm.at[jj_v[p]], zj_v[p], semGat[p]).wait()

        def issue_out(t, p):
            base = base0 + t * B
            pltpu.async_copy(r_v[p], r_hbm.at[pl.ds(base, B)], semOut[p])
            pltpu.async_copy(sh_v[p], sh_hbm.at[pl.ds(base * 16, B * 16)], semOut[p])

        def drain_out(p):
            pltpu.make_async_copy(r_v[p], r_hbm.at[pl.ds(0, B)], semOut[p]).wait()
            pltpu.make_async_copy(sh_v[p], sh_hbm.at[pl.ds(0, B * 16)], semOut[p]).wait()

        def compute(p):
            def grp(g, c):
                s = g * _LANES
                sl = pl.ds(s, _LANES)
                rows16 = (s + iota) * 16
                xi = xi_v[p][sl]
                yi = yi_v[p][sl]
                zi = zi_v[p][sl]
                xj = xj_v[p][sl]
                yj = yj_v[p][sl]
                zj = zj_v[p][sl]
                svx = sx_v[p][sl]
                svy = sy_v[p][sl]
                svz = sz_v[p][sl]
                dx = xj - xi + svx * c00 + svy * c10 + svz * c20
                dy = yj - yi + svx * c01 + svy * c11 + svz * c21
                dz = zj - zi + svx * c02 + svy * c12 + svz * c22
                r2 = dx * dx + dy * dy + dz * dz
                rinv = _rsqrt_newton(r2)
                r_v[p][sl] = r2 * rinv
                ux = dx * rinv
                uy = dy * rinv
                uz = dz * rinv
                sh = _sh_components(ux, uy, uz)
                for m in range(16):
                    plsc.store_scatter(sh_v[p], [rows16 + m], sh[m])
                return c
            lax.fori_loop(0, NG, grp, 0)

        issue_idx(0, 0)
        drain_idx(0)
        issue_gather(0)

        def step(t, p):
            q = 1 - p

            @pl.when(t < NBLK)
            def _():
                @pl.when(t + 1 < NBLK)
                def _():
                    issue_idx(t + 1, q)

                drain_gather(p)

                @pl.when(t + 1 < NBLK)
                def _():
                    drain_idx(q)
                    issue_gather(q)

                compute(p)
                issue_out(t, p)
                drain_out(p)

        def body2(o, carry):
            step(2 * o, 0)
            step(2 * o + 1, 1)
            return carry

        lax.fori_loop(0, (NBLK + 1) // 2, body2, 0)

    return sc_fn


def kernel(positions, cells, species, cell_shifts, centers, pairs,
           structure_centers, structure_pairs, structure_offsets):
    N = positions.shape[0]
    P = pairs.shape[0]
    px = positions[:, 0]
    py = positions[:, 1]
    pz = positions[:, 2]
    ii = pairs[:, 0]
    jj = pairs[:, 1]
    shifts_f = cell_shifts.astype(jnp.float32)
    sx = shifts_f[:, 0]
    sy = shifts_f[:, 1]
    sz = shifts_f[:, 2]
    cell16 = jnp.pad(cells[0].reshape(-1), (0, 7))
    off16 = jnp.pad(structure_offsets, (0, 16 - structure_offsets.shape[0]))
    r, sh_flat = _make_sc_fn(N, P)(px, py, pz, ii, jj, sx, sy, sz,
                                   cell16, off16)
    return (r, sh_flat.reshape(P, 16))
